# Initial kernel scaffold; baseline (speedup 1.0000x reference)
import jax
import jax.numpy as jnp
from jax.experimental import pallas as pl


def kernel(indices, table):
    raise NotImplementedError("write your pallas kernel here")


# speedup vs baseline: 1.1127x; 1.1127x over previous
import functools

import jax
import jax.numpy as jnp
from jax import lax
from jax.experimental import pallas as pl
from jax.experimental.pallas import tpu as pltpu
from jax.experimental.pallas import tpu_sc as plsc

NW = 32


def _build(N, V, D, per_w, C):
    n_chunks = per_w // C
    n_outer = n_chunks // 2
    mesh = plsc.VectorSubcoreMesh(core_axis_name="c", subcore_axis_name="s")

    @functools.partial(
        pl.kernel,
        mesh=mesh,
        out_type=jax.ShapeDtypeStruct((N, D), jnp.float32),
        scratch_types=[
            pltpu.VMEM((per_w,), jnp.int32),
            pltpu.VMEM((2, C, D), jnp.float32),
            pltpu.SemaphoreType.DMA,
            pltpu.SemaphoreType.DMA,
            pltpu.SemaphoreType.DMA,
            pltpu.SemaphoreType.DMA,
        ],
        compiler_params=pltpu.CompilerParams(use_tc_tiling_on_sc=False),
    )
    def k(idx_hbm, table_hbm, out_hbm, idx_v, rows_v, g0, g1, o0, o1):
        gsem = (g0, g1)
        osem = (o0, o1)
        wid = lax.axis_index("s") * 2 + lax.axis_index("c")
        base = wid * per_w
        pltpu.sync_copy(idx_hbm.at[pl.ds(base, per_w)], idx_v)
        for b in range(2):
            pltpu.async_copy(
                table_hbm.at[idx_v.at[pl.ds(b * C, C)]], rows_v.at[b], gsem[b]
            )

        def outer(i, _):
            for b in range(2):
                g = 2 * i + b
                pltpu.make_async_copy(
                    table_hbm.at[idx_v.at[pl.ds(0, C)]], rows_v.at[b], gsem[b]
                ).wait()
                pltpu.async_copy(
                    rows_v.at[b], out_hbm.at[pl.ds(base + g * C, C)], osem[b]
                )

                @pl.when(i < n_outer - 1)
                def _():
                    pltpu.make_async_copy(
                        rows_v.at[b], out_hbm.at[pl.ds(0, C)], osem[b]
                    ).wait()
                    pltpu.async_copy(
                        table_hbm.at[idx_v.at[pl.ds((g + 2) * C, C)]],
                        rows_v.at[b],
                        gsem[b],
                    )

            return 0

        lax.fori_loop(0, n_outer, outer, 0)
        for b in range(2):
            pltpu.make_async_copy(
                rows_v.at[b], out_hbm.at[pl.ds(0, C)], osem[b]
            ).wait()

    return k


def kernel(indices, table):
    B, H = indices.shape
    V, D = table.shape
    N = B * H
    per_w = N // NW
    C = 1280
    flat_idx = indices.reshape(N).astype(jnp.int32)
    out = _build(N, V, D, per_w, C)(flat_idx, table)
    return out.reshape(B, H, D)


# speedup vs baseline: 1.1137x; 1.0010x over previous
import functools

import jax
import jax.numpy as jnp
from jax import lax
from jax.experimental import pallas as pl
from jax.experimental.pallas import tpu as pltpu
from jax.experimental.pallas import tpu_sc as plsc

NW = 32
NBUF = 4


def _build(N, V, D, per_w, C):
    n_chunks = per_w // C
    n_outer = n_chunks // NBUF
    mesh = plsc.VectorSubcoreMesh(core_axis_name="c", subcore_axis_name="s")

    @functools.partial(
        pl.kernel,
        mesh=mesh,
        out_type=jax.ShapeDtypeStruct((N, D), jnp.float32),
        scratch_types=[
            pltpu.VMEM((per_w,), jnp.int32),
            pltpu.VMEM((NBUF, C, D), jnp.float32),
        ]
        + [pltpu.SemaphoreType.DMA] * (2 * NBUF),
        compiler_params=pltpu.CompilerParams(use_tc_tiling_on_sc=False),
    )
    def k(idx_hbm, table_hbm, out_hbm, idx_v, rows_v, *sems):
        gsem = sems[:NBUF]
        osem = sems[NBUF:]
        wid = lax.axis_index("s") * 2 + lax.axis_index("c")
        base = wid * per_w
        pltpu.sync_copy(idx_hbm.at[pl.ds(base, per_w)], idx_v)
        for b in range(NBUF):
            pltpu.async_copy(
                table_hbm.at[idx_v.at[pl.ds(b * C, C)]], rows_v.at[b], gsem[b]
            )

        def outer(i, _):
            for b in range(NBUF):
                g = NBUF * i + b
                pltpu.make_async_copy(
                    table_hbm.at[idx_v.at[pl.ds(0, C)]], rows_v.at[b], gsem[b]
                ).wait()
                pltpu.async_copy(
                    rows_v.at[b], out_hbm.at[pl.ds(base + g * C, C)], osem[b]
                )

                @pl.when(i < n_outer - 1)
                def _():
                    pltpu.make_async_copy(
                        rows_v.at[b], out_hbm.at[pl.ds(0, C)], osem[b]
                    ).wait()
                    pltpu.async_copy(
                        table_hbm.at[idx_v.at[pl.ds((g + NBUF) * C, C)]],
                        rows_v.at[b],
                        gsem[b],
                    )

            return 0

        lax.fori_loop(0, n_outer, outer, 0)
        for b in range(NBUF):
            pltpu.make_async_copy(
                rows_v.at[b], out_hbm.at[pl.ds(0, C)], osem[b]
            ).wait()

    return k


def kernel(indices, table):
    B, H = indices.shape
    V, D = table.shape
    N = B * H
    per_w = N // NW
    C = 640
    flat_idx = indices.reshape(N).astype(jnp.int32)
    out = _build(N, V, D, per_w, C)(flat_idx, table)
    return out.reshape(B, H, D)


# speedup vs baseline: 1.8142x; 1.6289x over previous
import functools

import jax
import jax.numpy as jnp
from jax import lax
from jax.experimental import pallas as pl
from jax.experimental.pallas import tpu as pltpu
from jax.experimental.pallas import tpu_sc as plsc

NW = 32
NBUF = 2


def _build(B, H, V, D, rows_w, CB):
    per_w = rows_w * H
    C = CB * H
    n_chunks = rows_w // CB
    n_outer = n_chunks // NBUF
    mesh = plsc.VectorSubcoreMesh(core_axis_name="c", subcore_axis_name="s")

    @functools.partial(
        pl.kernel,
        mesh=mesh,
        out_type=jax.ShapeDtypeStruct((B, H, D), jnp.float32),
        scratch_types=[
            pltpu.VMEM((per_w,), jnp.int32),
            pltpu.VMEM((NBUF, C, D), jnp.float32),
        ]
        + [pltpu.SemaphoreType.DMA] * (2 * NBUF),
        compiler_params=pltpu.CompilerParams(use_tc_tiling_on_sc=False),
    )
    def k(idx_hbm, table_hbm, out_hbm, idx_v, rows_v, *sems):
        gsem = sems[:NBUF]
        osem = sems[NBUF:]
        wid = lax.axis_index("s") * 2 + lax.axis_index("c")
        row0 = wid * rows_w
        base = wid * per_w
        pltpu.sync_copy(idx_hbm.at[pl.ds(base, per_w)], idx_v)
        for b in range(NBUF):
            pltpu.async_copy(
                table_hbm.at[idx_v.at[pl.ds(b * C, C)]], rows_v.at[b], gsem[b]
            )

        def outer(i, _):
            for b in range(NBUF):
                g = NBUF * i + b
                pltpu.make_async_copy(
                    table_hbm.at[idx_v.at[pl.ds(0, C)]], rows_v.at[b], gsem[b]
                ).wait()
                for r in range(CB):
                    pltpu.async_copy(
                        rows_v.at[b].at[pl.ds(r * H, H)],
                        out_hbm.at[row0 + g * CB + r],
                        osem[b],
                    )

                @pl.when(i < n_outer - 1)
                def _():
                    for r in range(CB):
                        pltpu.make_async_copy(
                            rows_v.at[b].at[pl.ds(r * H, H)],
                            out_hbm.at[row0],
                            osem[b],
                        ).wait()
                    pltpu.async_copy(
                        table_hbm.at[idx_v.at[pl.ds((g + NBUF) * C, C)]],
                        rows_v.at[b],
                        gsem[b],
                    )

            return 0

        lax.fori_loop(0, n_outer, outer, 0)
        for b in range(NBUF):
            for r in range(CB):
                pltpu.make_async_copy(
                    rows_v.at[b].at[pl.ds(r * H, H)],
                    out_hbm.at[row0],
                    osem[b],
                ).wait()

    return k


def kernel(indices, table):
    B, H = indices.shape
    V, D = table.shape
    N = B * H
    rows_w = B // NW
    CB = 32
    flat_idx = indices.reshape(N).astype(jnp.int32)
    return _build(B, H, V, D, rows_w, CB)(flat_idx, table)
